# baseline (speedup 1.0000x reference)
import functools

import jax
import jax.numpy as jnp
from jax import lax
from jax.experimental import pallas as pl
from jax.experimental.pallas import tpu as pltpu
from jax.experimental.pallas import tpu_sc as plsc

N, C, H, W = 4, 96, 384, 384
CP = 128
HW = H * W
NP = N * HW
NC, NS, L = 2, 16, 16
NW = NC * NS
PXW = NP // NW
P = 48
CHUNKS = PXW // P
GB = 12
GBP = GB * P
NB = 3


def _sc_grid_sample(xt, gxy):
    mesh = plsc.VectorSubcoreMesh(
        core_axis_name="c", subcore_axis_name="s", num_cores=NC,
        num_subcores=NS)

    scratch = (
        [pltpu.VMEM((2, GBP), jnp.float32)]
        + [pltpu.VMEM((P,), jnp.int32)] * (4 * NB)
        + [pltpu.VMEM((P,), jnp.float32)] * (4 * NB)
        + [pltpu.VMEM((P, CP), jnp.float32)] * (4 * NB)
        + [pltpu.VMEM((P, CP), jnp.float32)] * NB
        + [pltpu.SemaphoreType.DMA] * (2 * NB)
    )

    @functools.partial(
        pl.kernel,
        out_type=jax.ShapeDtypeStruct((NP, CP), jnp.float32),
        mesh=mesh,
        scratch_types=scratch,
        compiler_params=pltpu.CompilerParams(use_tc_tiling_on_sc=False),
    )
    def k(xt_hbm, gxy_hbm, out_hbm, gb_v, *rest):
        ii = [rest[4 * s:4 * s + 4] for s in range(NB)]
        o = 4 * NB
        ww = [rest[o + 4 * s:o + 4 * s + 4] for s in range(NB)]
        o = 8 * NB
        rr = [rest[o + 4 * s:o + 4 * s + 4] for s in range(NB)]
        o = 12 * NB
        ob = [rest[o + s] for s in range(NB)]
        o = 13 * NB
        sems = [rest[o + s] for s in range(NB)]
        osems = [rest[o + NB + s] for s in range(NB)]

        wid = lax.axis_index("s") * NC + lax.axis_index("c")
        px_base = wid * PXW
        batch_off = (px_base // HW) * HW

        zv = jnp.zeros((L,), jnp.float32)

        def zpad_body(p, c0):
            for s in range(NB):
                ob[s][p, pl.ds(C, L)] = zv
                ob[s][p, pl.ds(C + L, L)] = zv
            return c0

        lax.fori_loop(0, P, zpad_body, 0, unroll=False)

        def stage(g, s):
            pos = lax.rem(g, GB)

            @pl.when(pos == 0)
            def _():
                blk = px_base + g * P
                pltpu.sync_copy(gxy_hbm.at[pl.ds(blk, GBP)], gb_v.at[0])
                pltpu.sync_copy(gxy_hbm.at[pl.ds(NP + blk, GBP)], gb_v.at[1])

            off = pos * P
            for v in range(P // L):
                sl = pl.ds(off + v * L, L)
                so = pl.ds(v * L, L)
                gxv = gb_v[0, sl]
                gyv = gb_v[1, sl]
                ix = (gxv + 1.0) * (W * 0.5) - 0.5
                iy = (gyv + 1.0) * (H * 0.5) - 0.5
                tx = ix.astype(jnp.int32).astype(jnp.float32)
                ix0f = jnp.where(tx > ix, tx - 1.0, tx)
                ty = iy.astype(jnp.int32).astype(jnp.float32)
                iy0f = jnp.where(ty > iy, ty - 1.0, ty)
                wx1 = ix - ix0f
                wx0 = 1.0 - wx1
                wy1 = iy - iy0f
                wy0 = 1.0 - wy1
                ix0 = ix0f.astype(jnp.int32)
                ix1 = ix0 + 1
                iy0 = iy0f.astype(jnp.int32)
                iy1 = iy0 + 1
                vx0 = jnp.where((ix0 >= 0) & (ix0 < W), 1.0, 0.0)
                vx1 = jnp.where((ix1 >= 0) & (ix1 < W), 1.0, 0.0)
                vy0 = jnp.where((iy0 >= 0) & (iy0 < H), 1.0, 0.0)
                vy1 = jnp.where((iy1 >= 0) & (iy1 < H), 1.0, 0.0)
                xc0 = jnp.minimum(jnp.maximum(ix0, 0), W - 1)
                xc1 = jnp.minimum(jnp.maximum(ix1, 0), W - 1)
                yc0 = jnp.minimum(jnp.maximum(iy0, 0), H - 1)
                yc1 = jnp.minimum(jnp.maximum(iy1, 0), H - 1)
                r0 = yc0 * W + batch_off
                r1 = yc1 * W + batch_off
                ii[s][0][so] = r0 + xc0
                ii[s][1][so] = r0 + xc1
                ii[s][2][so] = r1 + xc0
                ii[s][3][so] = r1 + xc1
                ww[s][0][so] = wy0 * wx0 * vy0 * vx0
                ww[s][1][so] = wy0 * wx1 * vy0 * vx1
                ww[s][2][so] = wy1 * wx0 * vy1 * vx0
                ww[s][3][so] = wy1 * wx1 * vy1 * vx1
            for c in range(4):
                pltpu.async_copy(xt_hbm.at[ii[s][c]], rr[s][c], sems[s])

        def out_dst(g):
            return out_hbm.at[pl.ds(px_base + g * P, P)]

        def finish(g, s, first):
            for c in range(4):
                pltpu.make_async_copy(
                    xt_hbm.at[ii[s][c]], rr[s][c], sems[s]).wait()

            @pl.when(jnp.logical_not(first))
            def _():
                pltpu.make_async_copy(ob[s], out_dst(g), osems[s]).wait()

            r00_v, r01_v, r10_v, r11_v = rr[s]
            ob_v = ob[s]

            def grp_body(q, c2):
                qb = q * L
                sg = pl.ds(qb, L)
                wg00 = ww[s][0][sg]
                wg01 = ww[s][1][sg]
                wg10 = ww[s][2][sg]
                wg11 = ww[s][3][sg]
                for lane in range(L):
                    p = qb + lane
                    b00 = lax.broadcast(wg00[lane], (L,))
                    b01 = lax.broadcast(wg01[lane], (L,))
                    b10 = lax.broadcast(wg10[lane], (L,))
                    b11 = lax.broadcast(wg11[lane], (L,))
                    for j in range(C // L):
                        sj = pl.ds(j * L, L)
                        ob_v[p, sj] = (
                            (r00_v[p, sj] * b00 + r01_v[p, sj] * b01)
                            + (r10_v[p, sj] * b10 + r11_v[p, sj] * b11))
                return c2

            lax.fori_loop(0, P // L, grp_body, 0, unroll=False)
            pltpu.async_copy(ob_v, out_dst(g), osems[s])

        stage(0, 0)
        stage(1, 1)
        T = CHUNKS // NB

        def body(t, carry):
            g = NB * t
            stage(g + 2, 2)
            finish(g, 0, t == 0)

            @pl.when(t < T - 1)
            def _():
                stage(g + 3, 0)

            finish(g + 1, 1, t == 0)

            @pl.when(t < T - 1)
            def _():
                stage(g + 4, 1)

            finish(g + 2, 2, t == 0)
            return carry

        lax.fori_loop(0, T, body, 0, unroll=False)
        for s in range(NB):
            pltpu.make_async_copy(
                ob[s], out_dst(CHUNKS - NB + s), osems[s]).wait()

    return k(xt, gxy)


def kernel(x, grid):
    xt = jnp.pad(x.transpose(0, 2, 3, 1), ((0, 0), (0, 0), (0, 0), (0, CP - C))
                 ).reshape(NP, CP)
    gxy = grid.reshape(NP, 2).transpose(1, 0).reshape(2 * NP)
    out = _sc_grid_sample(xt, gxy)
    return out[:, :C].reshape(N, H, W, C).transpose(0, 3, 1, 2)
